# Initial kernel scaffold; baseline (speedup 1.0000x reference)
import jax
import jax.numpy as jnp
from jax.experimental import pallas as pl


def kernel(nodes, vis, h, c, W_in, b_in, W_rl, b_rl, W_rlv, b_rlv, W_score, b_score, W_mode, b_mode, W_ih, b_ih, W_hh, b_hh, W_pi, b_pi, W_mu, b_mu, W_sig, b_sig):
    raise NotImplementedError("write your pallas kernel here")


# speedup vs baseline: 2.9616x; 2.9616x over previous
import functools

import jax
import jax.numpy as jnp
from jax.experimental import pallas as pl

_RNN = 128
_K = 4
_G = 5
_OUT = 2
_BI = 16

_f32 = jnp.float32
_bf16 = jnp.bfloat16


def _social_lstm_kernel(nodes_ref, vis_ref, hrep_ref, h_ref, c_ref,
                        W_in_ref, b_in_ref, Wrl_ref, brl_ref, Wrlv_ref, brlv_ref,
                        Wexp_ref, Wmode_ref, bmode_ref,
                        W_ih_ref, b_ih_ref, W_hh_ref, b_hh_ref,
                        W_pi_ref, b_pi_ref, W_mu_ref, b_mu_ref, W_sig_ref, b_sig_ref,
                        pi_ref, mu_ref, sig_ref, h_out_ref, c_out_ref):
    i = pl.program_id(0)
    n = nodes_ref.shape[0]

    nodes = nodes_ref[...]
    nodes_i = nodes_ref[pl.ds(i * _BI, _BI), :]
    h_i = h_ref[...]
    c_i = c_ref[...]
    hrep_i = hrep_ref[...]

    def bdot(a, b_ref_val):
        return jnp.dot(a.astype(_bf16), b_ref_val, preferred_element_type=_f32)

    rela = (nodes_i[:, None, :] - nodes[None, :, :]).reshape(_BI * n, 6)
    rela_b = rela.astype(_bf16)

    loc = jax.nn.relu(jnp.dot(rela_b, Wrl_ref[...], preferred_element_type=_f32)
                      + brl_ref[...])
    prod = loc.reshape(_BI, n, _K * _RNN) * hrep_i[:, None, :]
    score = bdot(prod.reshape(_BI * n, _K * _RNN), Wexp_ref[...])

    smax = jnp.max(score, axis=-1, keepdims=True)
    iota_k = jax.lax.broadcasted_iota(jnp.int32, score.shape, 1)
    kfirst = jnp.min(jnp.where(score == smax, iota_k, _K), axis=-1, keepdims=True)
    onehot = (iota_k == kfirst).astype(_f32)

    locv = jax.nn.relu(jnp.dot(rela_b, Wrlv_ref[...], preferred_element_type=_f32)
                       + brlv_ref[...])
    cvec = (locv[:, 0 * _RNN:1 * _RNN] * onehot[:, 0:1]
            + locv[:, 1 * _RNN:2 * _RNN] * onehot[:, 1:2]
            + locv[:, 2 * _RNN:3 * _RNN] * onehot[:, 2:3]
            + locv[:, 3 * _RNN:4 * _RNN] * onehot[:, 3:4])
    mode = bdot(onehot, Wmode_ref[...]) + bmode_ref[...]
    combined = cvec + mode
    s = jnp.sum(combined, axis=-1).reshape(_BI, n)

    visf = vis_ref[...].astype(_f32)
    sel = onehot.reshape(_BI, n, _K) * visf[..., None]
    logits = jnp.where(sel > 0, s[..., None], _f32(-1e9))
    mj = jnp.max(logits, axis=1, keepdims=True)
    e = jnp.exp(logits - mj)
    p = e / jnp.sum(e, axis=1, keepdims=True)
    w = p * sel
    denom = jnp.clip(jnp.sum(w, axis=1, keepdims=True), 1e-9, None)
    w = w / denom
    any_sel = (jnp.sum(sel, axis=1, keepdims=True) > 0).astype(_f32)
    wtot = jnp.sum(w * any_sel, axis=-1)

    wb = wtot.astype(_bf16).astype(_f32)
    cb = combined.reshape(_BI, n, _RNN).astype(_bf16).astype(_f32)
    social = jnp.sum(wb[..., None] * cb, axis=1)

    inp_emb = jax.nn.relu(bdot(nodes_i[:, :4], W_in_ref[...]) + b_in_ref[...]) + social
    gates = (bdot(inp_emb, W_ih_ref[...]) + b_ih_ref[...]
             + bdot(h_i, W_hh_ref[...]) + b_hh_ref[...])
    i_g = gates[:, 0 * _RNN:1 * _RNN]
    f_g = gates[:, 1 * _RNN:2 * _RNN]
    g_g = gates[:, 2 * _RNN:3 * _RNN]
    o_g = gates[:, 3 * _RNN:4 * _RNN]
    c_new = jax.nn.sigmoid(f_g) * c_i + jax.nn.sigmoid(i_g) * jnp.tanh(g_g)
    h_new = jax.nn.sigmoid(o_g) * jnp.tanh(c_new)

    z = bdot(h_new, W_pi_ref[...]) + b_pi_ref[...]
    z = z - jnp.max(z, axis=1, keepdims=True)
    ez = jnp.exp(z)
    pi_ref[...] = ez / jnp.sum(ez, axis=1, keepdims=True)
    mu_ref[...] = bdot(h_new, W_mu_ref[...]) + b_mu_ref[...]
    sig_ref[...] = jnp.exp(bdot(h_new, W_sig_ref[...]) + b_sig_ref[...])
    h_out_ref[...] = h_new
    c_out_ref[...] = c_new


def _perm_kr(w):
    return w.reshape(w.shape[0], _RNN, _K).transpose(0, 2, 1).reshape(w.shape[0], _K * _RNN)


@jax.jit
def kernel(nodes, vis, h, c, W_in, b_in, W_rl, b_rl, W_rlv, b_rlv, W_score, b_score,
           W_mode, b_mode, W_ih, b_ih, W_hh, b_hh, W_pi, b_pi, W_mu, b_mu, W_sig, b_sig):
    n = nodes.shape[0]
    grid = (n // _BI,)

    Wrl_p = _perm_kr(W_rl).astype(_bf16)
    brl_p = b_rl.reshape(_RNN, _K).T.reshape(-1)
    Wrlv_p = _perm_kr(W_rlv).astype(_bf16)
    brlv_p = b_rlv.reshape(_RNN, _K).T.reshape(-1)
    hrep = jnp.tile(h, (1, _K))
    kr = _K * _RNN
    mask = (jnp.arange(kr)[:, None] // _RNN) == jnp.arange(_K)[None, :]
    Wexp = (jnp.tile(W_score[:, 0], _K)[:, None] * mask).astype(_bf16)

    Wmode_b = W_mode.astype(_bf16)
    W_in_b = W_in.astype(_bf16)
    W_ih_b = W_ih.astype(_bf16)
    W_hh_b = W_hh.astype(_bf16)
    W_pi_b = W_pi.astype(_bf16)
    W_mu_b = W_mu.astype(_bf16)
    W_sig_b = W_sig.astype(_bf16)

    full = lambda arr: pl.BlockSpec(arr.shape, lambda i: (0,) * arr.ndim)
    rows = lambda arr: pl.BlockSpec((_BI,) + arr.shape[1:], lambda i: (i,) + (0,) * (arr.ndim - 1))

    out_shapes = (
        jax.ShapeDtypeStruct((n, _G), _f32),
        jax.ShapeDtypeStruct((n, _G * _OUT), _f32),
        jax.ShapeDtypeStruct((n, _G * _OUT), _f32),
        jax.ShapeDtypeStruct((n, _RNN), _f32),
        jax.ShapeDtypeStruct((n, _RNN), _f32),
    )
    out_specs = tuple(rows(o) for o in out_shapes)

    args = (nodes, vis, hrep, h, c,
            W_in_b, b_in, Wrl_p, brl_p, Wrlv_p, brlv_p,
            Wexp, Wmode_b, b_mode,
            W_ih_b, b_ih, W_hh_b, b_hh,
            W_pi_b, b_pi, W_mu_b, b_mu, W_sig_b, b_sig)
    in_specs = [full(nodes), rows(vis), rows(hrep), rows(h), rows(c)] + [full(a) for a in args[5:]]

    pi, mu, sig, h_new, c_new = pl.pallas_call(
        _social_lstm_kernel,
        grid=grid,
        in_specs=in_specs,
        out_specs=out_specs,
        out_shape=out_shapes,
    )(*args)

    return pi, mu.reshape(n, _G, _OUT), sig.reshape(n, _G, _OUT), h_new, c_new


# speedup vs baseline: 2.9885x; 1.0091x over previous
import jax
import jax.numpy as jnp
from jax.experimental import pallas as pl

_RNN = 128
_K = 4
_G = 5
_OUT = 2
_BI = 16

_f32 = jnp.float32
_bf16 = jnp.bfloat16


def _social_lstm_kernel(nodes_a_ref, nodes_b_ref, vis_ref, hrep_ref, h_ref, c_ref,
                        W_in_ref, b_in_ref, Wrl7_ref, Wrlv7_ref,
                        Wexp_ref, B1_ref, modesum_ref, Wmode_ref, bmode_ref,
                        W_ih_ref, b_ih_ref, W_hh_ref, b_hh_ref,
                        W_pi_ref, b_pi_ref, W_mu_ref, b_mu_ref, W_sig_ref, b_sig_ref,
                        pi_ref, mu_ref, sig_ref, h_out_ref, c_out_ref):
    i = pl.program_id(0)
    n = nodes_b_ref.shape[0]

    nodes_a_i = nodes_a_ref[pl.ds(i * _BI, _BI), :]
    nodes_b = nodes_b_ref[...]
    h_i = h_ref[...]
    c_i = c_ref[...]
    hrep_i = hrep_ref[...]

    def bdot(a, b_val):
        return jnp.dot(a.astype(_bf16), b_val, preferred_element_type=_f32)

    rela = (nodes_a_i[:, None, :] - nodes_b[None, :, :]).reshape(_BI * n, 7)
    rela_b = rela.astype(_bf16)

    loc = jax.nn.relu(jnp.dot(rela_b, Wrl7_ref[...], preferred_element_type=_f32))
    prod = loc.reshape(_BI, n, _K * _RNN) * hrep_i[:, None, :]
    score = bdot(prod.reshape(_BI * n, _K * _RNN), Wexp_ref[...])

    smax = jnp.max(score, axis=-1, keepdims=True)
    kidx = jax.lax.broadcasted_iota(jnp.int32, score.shape, 1)
    pick = jnp.where(score == smax, -kidx, -_K)
    pmax = jnp.max(pick, axis=-1, keepdims=True)
    onehot = (pick == pmax).astype(_f32)

    locv = jax.nn.relu(jnp.dot(rela_b, Wrlv7_ref[...], preferred_element_type=_f32))
    cvec = (locv[:, 0 * _RNN:1 * _RNN] * onehot[:, 0:1]
            + locv[:, 1 * _RNN:2 * _RNN] * onehot[:, 1:2]
            + locv[:, 2 * _RNN:3 * _RNN] * onehot[:, 2:3]
            + locv[:, 3 * _RNN:4 * _RNN] * onehot[:, 3:4])

    s_all = (jnp.dot(locv, B1_ref[...], preferred_element_type=_f32)
             + modesum_ref[...]).reshape(_BI, n, _K)

    visf = vis_ref[...].astype(_f32)
    sel = onehot.reshape(_BI, n, _K) * visf[..., None]
    logits = jnp.where(sel > 0, s_all, _f32(-1e9))
    mj = jnp.max(logits, axis=1, keepdims=True)
    e = jnp.exp(logits - mj)
    p = e / jnp.sum(e, axis=1, keepdims=True)
    w = p * sel
    denom = jnp.clip(jnp.sum(w, axis=1, keepdims=True), 1e-9, None)
    any_sel = (jnp.sum(sel, axis=1, keepdims=True) > 0).astype(_f32)
    w = (w / denom) * any_sel

    wk = jnp.sum(w, axis=1)
    wtot = jnp.sum(w, axis=-1).reshape(_BI * n, 1)
    social = (jnp.sum((wtot * cvec).reshape(_BI, n, _RNN), axis=1)
              + jnp.dot(wk, Wmode_ref[...].astype(_f32), preferred_element_type=_f32)
              + jnp.sum(wk, axis=-1, keepdims=True) * bmode_ref[...])

    inp_emb = jax.nn.relu(bdot(nodes_a_i[:, :4], W_in_ref[...]) + b_in_ref[...]) + social
    gates = (bdot(inp_emb, W_ih_ref[...]) + b_ih_ref[...]
             + bdot(h_i, W_hh_ref[...]) + b_hh_ref[...])
    i_g = gates[:, 0 * _RNN:1 * _RNN]
    f_g = gates[:, 1 * _RNN:2 * _RNN]
    g_g = gates[:, 2 * _RNN:3 * _RNN]
    o_g = gates[:, 3 * _RNN:4 * _RNN]
    c_new = jax.nn.sigmoid(f_g) * c_i + jax.nn.sigmoid(i_g) * jnp.tanh(g_g)
    h_new = jax.nn.sigmoid(o_g) * jnp.tanh(c_new)

    z = bdot(h_new, W_pi_ref[...]) + b_pi_ref[...]
    z = z - jnp.max(z, axis=1, keepdims=True)
    ez = jnp.exp(z)
    pi_ref[...] = ez / jnp.sum(ez, axis=1, keepdims=True)
    mu_ref[...] = bdot(h_new, W_mu_ref[...]) + b_mu_ref[...]
    sig_ref[...] = jnp.exp(bdot(h_new, W_sig_ref[...]) + b_sig_ref[...])
    h_out_ref[...] = h_new
    c_out_ref[...] = c_new


def _perm_kr(w):
    return w.reshape(w.shape[0], _RNN, _K).transpose(0, 2, 1).reshape(w.shape[0], _K * _RNN)


@jax.jit
def kernel(nodes, vis, h, c, W_in, b_in, W_rl, b_rl, W_rlv, b_rlv, W_score, b_score,
           W_mode, b_mode, W_ih, b_ih, W_hh, b_hh, W_pi, b_pi, W_mu, b_mu, W_sig, b_sig):
    n = nodes.shape[0]
    grid = (n // _BI,)
    kr = _K * _RNN

    nodes_a = jnp.concatenate([nodes, jnp.ones((n, 1), _f32)], axis=1)
    nodes_b = jnp.concatenate([nodes, jnp.zeros((n, 1), _f32)], axis=1)
    Wrl7 = jnp.concatenate([_perm_kr(W_rl), b_rl.reshape(_RNN, _K).T.reshape(1, kr)],
                           axis=0).astype(_bf16)
    Wrlv7 = jnp.concatenate([_perm_kr(W_rlv), b_rlv.reshape(_RNN, _K).T.reshape(1, kr)],
                            axis=0).astype(_bf16)
    hrep = jnp.tile(h, (1, _K))
    blk = (jnp.arange(kr)[:, None] // _RNN) == jnp.arange(_K)[None, :]
    Wexp = (jnp.tile(W_score[:, 0], _K)[:, None] * blk).astype(_bf16)
    B1 = blk.astype(_f32)
    Wmode_b = W_mode.astype(_bf16)
    modesum = (Wmode_b.astype(_f32).sum(axis=1) + b_mode.sum())[None, :]

    W_in_b = W_in.astype(_bf16)
    W_ih_b = W_ih.astype(_bf16)
    W_hh_b = W_hh.astype(_bf16)
    W_pi_b = W_pi.astype(_bf16)
    W_mu_b = W_mu.astype(_bf16)
    W_sig_b = W_sig.astype(_bf16)

    full = lambda arr: pl.BlockSpec(arr.shape, lambda i: (0,) * arr.ndim)
    rows = lambda arr: pl.BlockSpec((_BI,) + arr.shape[1:], lambda i: (i,) + (0,) * (arr.ndim - 1))

    out_shapes = (
        jax.ShapeDtypeStruct((n, _G), _f32),
        jax.ShapeDtypeStruct((n, _G * _OUT), _f32),
        jax.ShapeDtypeStruct((n, _G * _OUT), _f32),
        jax.ShapeDtypeStruct((n, _RNN), _f32),
        jax.ShapeDtypeStruct((n, _RNN), _f32),
    )
    out_specs = tuple(rows(o) for o in out_shapes)

    args = (nodes_a, nodes_b, vis, hrep, h, c,
            W_in_b, b_in, Wrl7, Wrlv7,
            Wexp, B1, modesum, Wmode_b, b_mode,
            W_ih_b, b_ih, W_hh_b, b_hh,
            W_pi_b, b_pi, W_mu_b, b_mu, W_sig_b, b_sig)
    in_specs = ([full(nodes_a), full(nodes_b), rows(vis), rows(hrep), rows(h), rows(c)]
                + [full(a) for a in args[6:]])

    pi, mu, sig, h_new, c_new = pl.pallas_call(
        _social_lstm_kernel,
        grid=grid,
        in_specs=in_specs,
        out_specs=out_specs,
        out_shape=out_shapes,
    )(*args)

    return pi, mu.reshape(n, _G, _OUT), sig.reshape(n, _G, _OUT), h_new, c_new


# speedup vs baseline: 4.5433x; 1.5203x over previous
import jax
import jax.numpy as jnp
from jax.experimental import pallas as pl

_RNN = 128
_K = 4
_G = 5
_OUT = 2
_BI = 16

_f32 = jnp.float32
_bf16 = jnp.bfloat16


def _social_lstm_kernel(nodes_ref, vis_ref, h_ref, c_ref,
                        W_in_ref, b_in_ref, Wrl7_ref, Wrlv7_ref, Wsc_ref,
                        Wmode_ref, bmode_ref,
                        W_ih_ref, b_ih_ref, W_hh_ref, b_hh_ref,
                        W_pi_ref, b_pi_ref, W_mu_ref, b_mu_ref, W_sig_ref, b_sig_ref,
                        pi_ref, mu_ref, sig_ref, h_out_ref, c_out_ref):
    i = pl.program_id(0)
    n = nodes_ref.shape[0]
    m = _BI * n

    nodes = nodes_ref[...]
    nodes_i = nodes_ref[pl.ds(i * _BI, _BI), :]
    h_i = h_ref[...]
    c_i = c_ref[...]

    def bdot(a, b_val):
        return jnp.dot(a.astype(_bf16), b_val.astype(_bf16),
                       preferred_element_type=_f32)

    rela = (nodes_i[:, None, :] - nodes[None, :, :]).reshape(m, 6)
    rela7 = jnp.concatenate([rela, jnp.ones((m, 1), _f32)], axis=1).astype(_bf16)

    loc = jax.nn.relu(jnp.dot(rela7, Wrl7_ref[...], preferred_element_type=_f32))
    locv = jax.nn.relu(jnp.dot(rela7, Wrlv7_ref[...], preferred_element_type=_f32))
    hrep_i = jnp.concatenate([h_i] * _K, axis=1)
    prod = (loc.reshape(_BI, n, _K * _RNN) * hrep_i[:, None, :]).reshape(m, _K * _RNN)

    wsc = Wsc_ref[...].astype(_bf16)
    prod_b = prod.astype(_bf16)
    ones_r = jnp.ones((_RNN, 1), _f32)
    cols = [jnp.dot(prod_b[:, k * _RNN:(k + 1) * _RNN], wsc,
                    preferred_element_type=_f32) for k in range(_K)]
    cols += [jnp.dot(locv[:, k * _RNN:(k + 1) * _RNN], ones_r,
                     preferred_element_type=_f32) for k in range(_K)]
    st = jnp.concatenate(cols, axis=1)
    st = st.T.reshape(2 * _K, _BI, n)
    score_t = st[:_K]
    ssum_t = st[_K:]

    smax = jnp.max(score_t, axis=0, keepdims=True)
    kidx = jax.lax.broadcasted_iota(jnp.int32, score_t.shape, 0)
    pick = jnp.where(score_t == smax, -kidx, -_K)
    pmax = jnp.max(pick, axis=0, keepdims=True)
    onehot_t = (pick == pmax).astype(_f32)

    visf = vis_ref[...].astype(_f32)
    sel = onehot_t * visf[None]
    logits = jnp.where(sel > 0, ssum_t, _f32(-1e9))
    mj = jnp.max(logits, axis=2, keepdims=True)
    e = jnp.exp(logits - mj)
    p = e / jnp.sum(e, axis=2, keepdims=True)
    w = p * sel
    denom = jnp.clip(jnp.sum(w, axis=2, keepdims=True), 1e-9, None)
    any_sel = (jnp.sum(sel, axis=2, keepdims=True) > 0).astype(_f32)
    w = (w / denom) * any_sel

    wmode_b = Wmode_ref[...].astype(_bf16).astype(_f32)
    modeflat = (jnp.concatenate([wmode_b[k][None, :] for k in range(_K)], axis=1)
                + jnp.concatenate([bmode_ref[...][None, :]] * _K, axis=1))
    locvm = (locv + modeflat).astype(_bf16)
    w_b = jnp.transpose(w, (1, 0, 2)).astype(_bf16)
    locvm3 = locvm.reshape(_BI, n, _K * _RNN)
    soc4 = jax.lax.dot_general(w_b, locvm3, (((2,), (1,)), ((0,), (0,))),
                               preferred_element_type=_f32)
    social = (soc4[:, 0, 0 * _RNN:1 * _RNN] + soc4[:, 1, 1 * _RNN:2 * _RNN]
              + soc4[:, 2, 2 * _RNN:3 * _RNN] + soc4[:, 3, 3 * _RNN:4 * _RNN])

    inp_emb = jax.nn.relu(bdot(nodes_i[:, :4], W_in_ref[...]) + b_in_ref[...]) + social
    gates = (bdot(inp_emb, W_ih_ref[...]) + b_ih_ref[...]
             + bdot(h_i, W_hh_ref[...]) + b_hh_ref[...])
    i_g = gates[:, 0 * _RNN:1 * _RNN]
    f_g = gates[:, 1 * _RNN:2 * _RNN]
    g_g = gates[:, 2 * _RNN:3 * _RNN]
    o_g = gates[:, 3 * _RNN:4 * _RNN]
    c_new = jax.nn.sigmoid(f_g) * c_i + jax.nn.sigmoid(i_g) * jnp.tanh(g_g)
    h_new = jax.nn.sigmoid(o_g) * jnp.tanh(c_new)

    z = bdot(h_new, W_pi_ref[...]) + b_pi_ref[...]
    z = z - jnp.max(z, axis=1, keepdims=True)
    ez = jnp.exp(z)
    pi_ref[...] = ez / jnp.sum(ez, axis=1, keepdims=True)
    mu_ref[...] = bdot(h_new, W_mu_ref[...]) + b_mu_ref[...]
    sig_ref[...] = jnp.exp(bdot(h_new, W_sig_ref[...]) + b_sig_ref[...])
    h_out_ref[...] = h_new
    c_out_ref[...] = c_new


def _perm_kr(w):
    return w.reshape(w.shape[0], _RNN, _K).transpose(0, 2, 1).reshape(w.shape[0], _K * _RNN)


@jax.jit
def kernel(nodes, vis, h, c, W_in, b_in, W_rl, b_rl, W_rlv, b_rlv, W_score, b_score,
           W_mode, b_mode, W_ih, b_ih, W_hh, b_hh, W_pi, b_pi, W_mu, b_mu, W_sig, b_sig):
    n = nodes.shape[0]
    grid = (n // _BI,)
    kr = _K * _RNN

    Wrl7 = jnp.concatenate([_perm_kr(W_rl), b_rl.reshape(_RNN, _K).T.reshape(1, kr)],
                           axis=0).astype(_bf16)
    Wrlv7 = jnp.concatenate([_perm_kr(W_rlv), b_rlv.reshape(_RNN, _K).T.reshape(1, kr)],
                            axis=0).astype(_bf16)

    full = lambda arr: pl.BlockSpec(arr.shape, lambda i: (0,) * arr.ndim)
    rows = lambda arr: pl.BlockSpec((_BI,) + arr.shape[1:], lambda i: (i,) + (0,) * (arr.ndim - 1))

    out_shapes = (
        jax.ShapeDtypeStruct((n, _G), _f32),
        jax.ShapeDtypeStruct((n, _G * _OUT), _f32),
        jax.ShapeDtypeStruct((n, _G * _OUT), _f32),
        jax.ShapeDtypeStruct((n, _RNN), _f32),
        jax.ShapeDtypeStruct((n, _RNN), _f32),
    )
    out_specs = tuple(rows(o) for o in out_shapes)

    args = (nodes, vis, h, c,
            W_in, b_in, Wrl7, Wrlv7, W_score,
            W_mode, b_mode,
            W_ih, b_ih, W_hh, b_hh,
            W_pi, b_pi, W_mu, b_mu, W_sig, b_sig)
    in_specs = [full(nodes), rows(vis), rows(h), rows(c)] + [full(a) for a in args[4:]]

    pi, mu, sig, h_new, c_new = pl.pallas_call(
        _social_lstm_kernel,
        grid=grid,
        in_specs=in_specs,
        out_specs=out_specs,
        out_shape=out_shapes,
    )(*args)

    return pi, mu.reshape(n, _G, _OUT), sig.reshape(n, _G, _OUT), h_new, c_new


# speedup vs baseline: 5.5610x; 1.2240x over previous
import jax
import jax.numpy as jnp
from jax.experimental import pallas as pl

_RNN = 128
_K = 4
_G = 5
_OUT = 2
_BI = 16

_f32 = jnp.float32
_bf16 = jnp.bfloat16


def _social_lstm_kernel(nodes_ref, vis_ref, h_ref, c_ref,
                        W_in_ref, b_in_ref, Wrl7_ref, Wrlv7_ref, Wsc_ref,
                        Wmode_ref, bmode_ref,
                        W_ih_ref, b_ih_ref, W_hh_ref, b_hh_ref,
                        W_pi_ref, b_pi_ref, W_mu_ref, b_mu_ref, W_sig_ref, b_sig_ref,
                        pi_ref, mu_ref, sig_ref, h_out_ref, c_out_ref):
    i = pl.program_id(0)
    n = nodes_ref.shape[0]
    m = _BI * n

    nodes = nodes_ref[...]
    nodes_i = nodes_ref[pl.ds(i * _BI, _BI), :]
    h_i = h_ref[...]
    c_i = c_ref[...]

    def bdot(a, b_val):
        return jnp.dot(a.astype(_bf16), b_val.astype(_bf16),
                       preferred_element_type=_f32)

    rela = (nodes_i[:, None, :] - nodes[None, :, :]).reshape(m, 6)
    rela7 = jnp.concatenate([rela, jnp.ones((m, 1), _f32)], axis=1).astype(_bf16)

    loc = jax.nn.relu(jnp.dot(rela7, Wrl7_ref[...], preferred_element_type=_f32))
    locv = jax.nn.relu(jnp.dot(rela7, Wrlv7_ref[...], preferred_element_type=_f32))
    hrep_i = jnp.concatenate([h_i] * _K, axis=1)
    prod = (loc.reshape(_BI, n, _K * _RNN) * hrep_i[:, None, :]).reshape(m, _K * _RNN)

    kr = _K * _RNN
    blk = (jax.lax.broadcasted_iota(jnp.int32, (kr, _K), 0) // _RNN
           == jax.lax.broadcasted_iota(jnp.int32, (kr, _K), 1))
    wsc_t = jnp.concatenate([Wsc_ref[...]] * _K, axis=0)
    wexp = jnp.where(blk, wsc_t, _f32(0.0)).astype(_bf16)
    b1 = blk.astype(_f32)
    score_c = jnp.dot(prod.astype(_bf16), wexp, preferred_element_type=_f32)
    ssum_c = jnp.dot(locv, b1, preferred_element_type=_f32)
    st = jnp.concatenate([score_c, ssum_c], axis=1)
    st = st.T.reshape(2 * _K, _BI, n)
    score_t = st[:_K]
    ssum_t = st[_K:]

    smax = jnp.max(score_t, axis=0, keepdims=True)
    kidx = jax.lax.broadcasted_iota(jnp.int32, score_t.shape, 0)
    pick = jnp.where(score_t == smax, -kidx, -_K)
    pmax = jnp.max(pick, axis=0, keepdims=True)
    onehot_t = (pick == pmax).astype(_f32)

    visf = vis_ref[...].astype(_f32)
    sel = onehot_t * visf[None]
    logits = jnp.where(sel > 0, ssum_t, _f32(-1e9))
    mj = jnp.max(logits, axis=2, keepdims=True)
    e = jnp.exp(logits - mj)
    p = e / jnp.sum(e, axis=2, keepdims=True)
    w = p * sel
    denom = jnp.clip(jnp.sum(w, axis=2, keepdims=True), 1e-9, None)
    any_sel = (jnp.sum(sel, axis=2, keepdims=True) > 0).astype(_f32)
    w = (w / denom) * any_sel

    wmode_b = Wmode_ref[...].astype(_bf16).astype(_f32)
    modeflat = (jnp.concatenate([wmode_b[k][None, :] for k in range(_K)], axis=1)
                + jnp.concatenate([bmode_ref[...][None, :]] * _K, axis=1))
    locvm = (locv + modeflat).astype(_bf16)
    w_b = jnp.transpose(w, (1, 0, 2)).astype(_bf16)
    locvm3 = locvm.reshape(_BI, n, _K * _RNN)
    soc4 = jax.lax.dot_general(w_b, locvm3, (((2,), (1,)), ((0,), (0,))),
                               preferred_element_type=_f32)
    social = (soc4[:, 0, 0 * _RNN:1 * _RNN] + soc4[:, 1, 1 * _RNN:2 * _RNN]
              + soc4[:, 2, 2 * _RNN:3 * _RNN] + soc4[:, 3, 3 * _RNN:4 * _RNN])

    inp_emb = jax.nn.relu(bdot(nodes_i[:, :4], W_in_ref[...]) + b_in_ref[...]) + social
    gates = (bdot(inp_emb, W_ih_ref[...]) + b_ih_ref[...]
             + bdot(h_i, W_hh_ref[...]) + b_hh_ref[...])
    i_g = gates[:, 0 * _RNN:1 * _RNN]
    f_g = gates[:, 1 * _RNN:2 * _RNN]
    g_g = gates[:, 2 * _RNN:3 * _RNN]
    o_g = gates[:, 3 * _RNN:4 * _RNN]
    c_new = jax.nn.sigmoid(f_g) * c_i + jax.nn.sigmoid(i_g) * jnp.tanh(g_g)
    h_new = jax.nn.sigmoid(o_g) * jnp.tanh(c_new)

    z = bdot(h_new, W_pi_ref[...]) + b_pi_ref[...]
    z = z - jnp.max(z, axis=1, keepdims=True)
    ez = jnp.exp(z)
    pi_ref[...] = ez / jnp.sum(ez, axis=1, keepdims=True)
    mu_ref[...] = bdot(h_new, W_mu_ref[...]) + b_mu_ref[...]
    sig_ref[...] = jnp.exp(bdot(h_new, W_sig_ref[...]) + b_sig_ref[...])
    h_out_ref[...] = h_new
    c_out_ref[...] = c_new


def _perm_kr(w):
    return w.reshape(w.shape[0], _RNN, _K).transpose(0, 2, 1).reshape(w.shape[0], _K * _RNN)


@jax.jit
def kernel(nodes, vis, h, c, W_in, b_in, W_rl, b_rl, W_rlv, b_rlv, W_score, b_score,
           W_mode, b_mode, W_ih, b_ih, W_hh, b_hh, W_pi, b_pi, W_mu, b_mu, W_sig, b_sig):
    n = nodes.shape[0]
    grid = (n // _BI,)
    kr = _K * _RNN

    Wrl7 = jnp.concatenate([_perm_kr(W_rl), b_rl.reshape(_RNN, _K).T.reshape(1, kr)],
                           axis=0).astype(_bf16)
    Wrlv7 = jnp.concatenate([_perm_kr(W_rlv), b_rlv.reshape(_RNN, _K).T.reshape(1, kr)],
                            axis=0).astype(_bf16)

    full = lambda arr: pl.BlockSpec(arr.shape, lambda i: (0,) * arr.ndim)
    rows = lambda arr: pl.BlockSpec((_BI,) + arr.shape[1:], lambda i: (i,) + (0,) * (arr.ndim - 1))

    out_shapes = (
        jax.ShapeDtypeStruct((n, _G), _f32),
        jax.ShapeDtypeStruct((n, _G * _OUT), _f32),
        jax.ShapeDtypeStruct((n, _G * _OUT), _f32),
        jax.ShapeDtypeStruct((n, _RNN), _f32),
        jax.ShapeDtypeStruct((n, _RNN), _f32),
    )
    out_specs = tuple(rows(o) for o in out_shapes)

    args = (nodes, vis, h, c,
            W_in, b_in, Wrl7, Wrlv7, W_score,
            W_mode, b_mode,
            W_ih, b_ih, W_hh, b_hh,
            W_pi, b_pi, W_mu, b_mu, W_sig, b_sig)
    in_specs = [full(nodes), rows(vis), rows(h), rows(c)] + [full(a) for a in args[4:]]

    pi, mu, sig, h_new, c_new = pl.pallas_call(
        _social_lstm_kernel,
        grid=grid,
        in_specs=in_specs,
        out_specs=out_specs,
        out_shape=out_shapes,
    )(*args)

    return pi, mu.reshape(n, _G, _OUT), sig.reshape(n, _G, _OUT), h_new, c_new


# speedup vs baseline: 6.0675x; 1.0911x over previous
import jax
import jax.numpy as jnp
from jax.experimental import pallas as pl

_RNN = 128
_K = 4
_G = 5
_OUT = 2
_BI = 32

_f32 = jnp.float32
_bf16 = jnp.bfloat16


def _social_lstm_kernel(nodes_ref, vis_ref, h_ref, c_ref,
                        W_in_ref, b_in_ref, Wrl7_ref, Wrlv7_ref, Wsc_ref,
                        Wmode_ref, bmode_ref,
                        W_ih_ref, b_ih_ref, W_hh_ref, b_hh_ref,
                        W_pi_ref, b_pi_ref, W_mu_ref, b_mu_ref, W_sig_ref, b_sig_ref,
                        pi_ref, mu_ref, sig_ref, h_out_ref, c_out_ref):
    i = pl.program_id(0)
    n = nodes_ref.shape[0]
    m = _BI * n

    nodes = nodes_ref[...]
    nodes_i = nodes_ref[pl.ds(i * _BI, _BI), :]
    h_i = h_ref[...]
    c_i = c_ref[...]

    def bdot(a, b_val):
        return jnp.dot(a.astype(_bf16), b_val.astype(_bf16),
                       preferred_element_type=_f32)

    rela = (nodes_i[:, None, :] - nodes[None, :, :]).reshape(m, 6)
    rela7 = jnp.concatenate([rela, jnp.ones((m, 1), _f32)], axis=1).astype(_bf16)

    loc = jax.nn.relu(jnp.dot(rela7, Wrl7_ref[...], preferred_element_type=_f32))
    locv = jax.nn.relu(jnp.dot(rela7, Wrlv7_ref[...], preferred_element_type=_f32))
    hrep_i = jnp.concatenate([h_i] * _K, axis=1)
    prod = (loc.reshape(_BI, n, _K * _RNN) * hrep_i[:, None, :]).reshape(m, _K * _RNN)

    kr = _K * _RNN
    blk = (jax.lax.broadcasted_iota(jnp.int32, (kr, _K), 0) // _RNN
           == jax.lax.broadcasted_iota(jnp.int32, (kr, _K), 1))
    wsc_t = jnp.concatenate([Wsc_ref[...]] * _K, axis=0)
    wexp = jnp.where(blk, wsc_t, _f32(0.0)).astype(_bf16)
    b1 = blk.astype(_f32)
    score_c = jnp.dot(prod.astype(_bf16), wexp, preferred_element_type=_f32)
    ssum_c = jnp.dot(locv, b1, preferred_element_type=_f32)
    st = jnp.concatenate([score_c, ssum_c], axis=1)
    st = st.T.reshape(2 * _K, _BI, n)
    score_t = st[:_K]
    ssum_t = st[_K:]

    smax = jnp.max(score_t, axis=0, keepdims=True)
    kidx = jax.lax.broadcasted_iota(jnp.int32, score_t.shape, 0)
    pick = jnp.where(score_t == smax, -kidx, -_K)
    pmax = jnp.max(pick, axis=0, keepdims=True)
    onehot_t = (pick == pmax).astype(_f32)

    visf = vis_ref[...].astype(_f32)
    sel = onehot_t * visf[None]
    logits = jnp.where(sel > 0, ssum_t, _f32(-1e9))
    mj = jnp.max(logits, axis=2, keepdims=True)
    e = jnp.exp(logits - mj)
    p = e / jnp.sum(e, axis=2, keepdims=True)
    w = p * sel
    denom = jnp.clip(jnp.sum(w, axis=2, keepdims=True), 1e-9, None)
    any_sel = (jnp.sum(sel, axis=2, keepdims=True) > 0).astype(_f32)
    w = (w / denom) * any_sel

    wmode_b = Wmode_ref[...].astype(_bf16).astype(_f32)
    modeflat = (jnp.concatenate([wmode_b[k][None, :] for k in range(_K)], axis=1)
                + jnp.concatenate([bmode_ref[...][None, :]] * _K, axis=1))
    locvm = (locv + modeflat).astype(_bf16)
    w_b = jnp.transpose(w, (1, 0, 2)).astype(_bf16)
    locvm3 = locvm.reshape(_BI, n, _K * _RNN)
    soc4 = jax.lax.dot_general(w_b, locvm3, (((2,), (1,)), ((0,), (0,))),
                               preferred_element_type=_f32)
    social = (soc4[:, 0, 0 * _RNN:1 * _RNN] + soc4[:, 1, 1 * _RNN:2 * _RNN]
              + soc4[:, 2, 2 * _RNN:3 * _RNN] + soc4[:, 3, 3 * _RNN:4 * _RNN])

    inp_emb = jax.nn.relu(bdot(nodes_i[:, :4], W_in_ref[...]) + b_in_ref[...]) + social
    gates = (bdot(inp_emb, W_ih_ref[...]) + b_ih_ref[...]
             + bdot(h_i, W_hh_ref[...]) + b_hh_ref[...])
    i_g = gates[:, 0 * _RNN:1 * _RNN]
    f_g = gates[:, 1 * _RNN:2 * _RNN]
    g_g = gates[:, 2 * _RNN:3 * _RNN]
    o_g = gates[:, 3 * _RNN:4 * _RNN]
    c_new = jax.nn.sigmoid(f_g) * c_i + jax.nn.sigmoid(i_g) * jnp.tanh(g_g)
    h_new = jax.nn.sigmoid(o_g) * jnp.tanh(c_new)

    z = bdot(h_new, W_pi_ref[...]) + b_pi_ref[...]
    z = z - jnp.max(z, axis=1, keepdims=True)
    ez = jnp.exp(z)
    pi_ref[...] = ez / jnp.sum(ez, axis=1, keepdims=True)
    mu_ref[...] = bdot(h_new, W_mu_ref[...]) + b_mu_ref[...]
    sig_ref[...] = jnp.exp(bdot(h_new, W_sig_ref[...]) + b_sig_ref[...])
    h_out_ref[...] = h_new
    c_out_ref[...] = c_new


def _perm_kr(w):
    return w.reshape(w.shape[0], _RNN, _K).transpose(0, 2, 1).reshape(w.shape[0], _K * _RNN)


@jax.jit
def kernel(nodes, vis, h, c, W_in, b_in, W_rl, b_rl, W_rlv, b_rlv, W_score, b_score,
           W_mode, b_mode, W_ih, b_ih, W_hh, b_hh, W_pi, b_pi, W_mu, b_mu, W_sig, b_sig):
    n = nodes.shape[0]
    grid = (n // _BI,)
    kr = _K * _RNN

    Wrl7 = jnp.concatenate([_perm_kr(W_rl), b_rl.reshape(_RNN, _K).T.reshape(1, kr)],
                           axis=0).astype(_bf16)
    Wrlv7 = jnp.concatenate([_perm_kr(W_rlv), b_rlv.reshape(_RNN, _K).T.reshape(1, kr)],
                            axis=0).astype(_bf16)

    full = lambda arr: pl.BlockSpec(arr.shape, lambda i: (0,) * arr.ndim)
    rows = lambda arr: pl.BlockSpec((_BI,) + arr.shape[1:], lambda i: (i,) + (0,) * (arr.ndim - 1))

    out_shapes = (
        jax.ShapeDtypeStruct((n, _G), _f32),
        jax.ShapeDtypeStruct((n, _G * _OUT), _f32),
        jax.ShapeDtypeStruct((n, _G * _OUT), _f32),
        jax.ShapeDtypeStruct((n, _RNN), _f32),
        jax.ShapeDtypeStruct((n, _RNN), _f32),
    )
    out_specs = tuple(rows(o) for o in out_shapes)

    args = (nodes, vis, h, c,
            W_in, b_in, Wrl7, Wrlv7, W_score,
            W_mode, b_mode,
            W_ih, b_ih, W_hh, b_hh,
            W_pi, b_pi, W_mu, b_mu, W_sig, b_sig)
    in_specs = [full(nodes), rows(vis), rows(h), rows(c)] + [full(a) for a in args[4:]]

    pi, mu, sig, h_new, c_new = pl.pallas_call(
        _social_lstm_kernel,
        grid=grid,
        in_specs=in_specs,
        out_specs=out_specs,
        out_shape=out_shapes,
    )(*args)

    return pi, mu.reshape(n, _G, _OUT), sig.reshape(n, _G, _OUT), h_new, c_new


# speedup vs baseline: 6.3818x; 1.0518x over previous
import jax
import jax.numpy as jnp
from jax.experimental import pallas as pl
from jax.experimental.pallas import tpu as pltpu

_RNN = 128
_K = 4
_G = 5
_OUT = 2
_BI = 32

_f32 = jnp.float32
_bf16 = jnp.bfloat16


def _social_lstm_kernel(nodes_ref, vis_ref, h_ref, c_ref,
                        W_in_ref, b_in_ref, Wrl_ref, brl_ref, Wrlv_ref, brlv_ref,
                        Wsc_ref, Wmode_ref, bmode_ref,
                        W_ih_ref, b_ih_ref, W_hh_ref, b_hh_ref,
                        W_pi_ref, b_pi_ref, W_mu_ref, b_mu_ref, W_sig_ref, b_sig_ref,
                        pi_ref, mu_ref, sig_ref, h_out_ref, c_out_ref,
                        w7_ref):
    i = pl.program_id(0)
    n = nodes_ref.shape[0]
    m = _BI * n
    kr = _K * _RNN

    nodes = nodes_ref[...]
    nodes_i = nodes_ref[pl.ds(i * _BI, _BI), :]
    h_i = h_ref[...]
    c_i = c_ref[...]

    def bdot(a, b_val):
        return jnp.dot(a.astype(_bf16), b_val.astype(_bf16),
                       preferred_element_type=_f32)

    @pl.when(i == 0)
    def _prep():
        src = jax.lax.broadcasted_iota(jnp.int32, (kr, kr), 0)
        dst = jax.lax.broadcasted_iota(jnp.int32, (kr, kr), 1)
        P = ((src % _K) * _RNN + src // _K == dst).astype(_bf16)
        wa = jnp.concatenate([Wrl_ref[...], brl_ref[...][None, :]], axis=0)
        wb = jnp.concatenate([Wrlv_ref[...], brlv_ref[...][None, :]], axis=0)
        w2 = jnp.concatenate([wa, wb], axis=1).astype(_bf16)
        w7_ref[:, 0 * kr:1 * kr] = jnp.dot(w2[:, 0 * kr:1 * kr], P,
                                           preferred_element_type=_f32).astype(_bf16)
        w7_ref[:, 1 * kr:2 * kr] = jnp.dot(w2[:, 1 * kr:2 * kr], P,
                                           preferred_element_type=_f32).astype(_bf16)

    rela = (nodes_i[:, None, :] - nodes[None, :, :]).reshape(m, 6)
    rela7 = jnp.concatenate([rela, jnp.ones((m, 1), _f32)], axis=1).astype(_bf16)

    locb = jnp.dot(rela7, w7_ref[...], preferred_element_type=_f32)
    loc = jax.nn.relu(locb[:, 0 * kr:1 * kr])
    locv = jax.nn.relu(locb[:, 1 * kr:2 * kr])
    hrep_i = jnp.concatenate([h_i] * _K, axis=1)
    prod = (loc.reshape(_BI, n, _K * _RNN) * hrep_i[:, None, :]).reshape(m, _K * _RNN)

    blk = (jax.lax.broadcasted_iota(jnp.int32, (kr, _K), 0) // _RNN
           == jax.lax.broadcasted_iota(jnp.int32, (kr, _K), 1))
    wsc_t = jnp.concatenate([Wsc_ref[...]] * _K, axis=0)
    wexp = jnp.where(blk, wsc_t, _f32(0.0)).astype(_bf16)
    b1 = blk.astype(_f32)
    score_c = jnp.dot(prod.astype(_bf16), wexp, preferred_element_type=_f32)
    ssum_c = jnp.dot(locv, b1, preferred_element_type=_f32)
    st = jnp.concatenate([score_c, ssum_c], axis=1)
    st = st.T.reshape(2 * _K, _BI, n)
    score_t = st[:_K]
    ssum_t = st[_K:]

    smax = jnp.max(score_t, axis=0, keepdims=True)
    kidx = jax.lax.broadcasted_iota(jnp.int32, score_t.shape, 0)
    pick = jnp.where(score_t == smax, -kidx, -_K)
    pmax = jnp.max(pick, axis=0, keepdims=True)
    onehot_t = (pick == pmax).astype(_f32)

    visf = vis_ref[...].astype(_f32)
    sel = onehot_t * visf[None]
    logits = jnp.where(sel > 0, ssum_t, _f32(-1e9))
    mj = jnp.max(logits, axis=2, keepdims=True)
    e = jnp.exp(logits - mj)
    p = e / jnp.sum(e, axis=2, keepdims=True)
    w = p * sel
    denom = jnp.clip(jnp.sum(w, axis=2, keepdims=True), 1e-9, None)
    any_sel = (jnp.sum(sel, axis=2, keepdims=True) > 0).astype(_f32)
    w = (w / denom) * any_sel

    wmode_b = Wmode_ref[...].astype(_bf16).astype(_f32)
    modeflat = (jnp.concatenate([wmode_b[k][None, :] for k in range(_K)], axis=1)
                + jnp.concatenate([bmode_ref[...][None, :]] * _K, axis=1))
    locvm = (locv + modeflat).astype(_bf16)
    w_b = jnp.transpose(w, (1, 0, 2)).astype(_bf16)
    locvm3 = locvm.reshape(_BI, n, _K * _RNN)
    soc4 = jax.lax.dot_general(w_b, locvm3, (((2,), (1,)), ((0,), (0,))),
                               preferred_element_type=_f32)
    social = (soc4[:, 0, 0 * _RNN:1 * _RNN] + soc4[:, 1, 1 * _RNN:2 * _RNN]
              + soc4[:, 2, 2 * _RNN:3 * _RNN] + soc4[:, 3, 3 * _RNN:4 * _RNN])

    inp_emb = jax.nn.relu(bdot(nodes_i[:, :4], W_in_ref[...]) + b_in_ref[...]) + social
    gates = (bdot(inp_emb, W_ih_ref[...]) + b_ih_ref[...]
             + bdot(h_i, W_hh_ref[...]) + b_hh_ref[...])
    i_g = gates[:, 0 * _RNN:1 * _RNN]
    f_g = gates[:, 1 * _RNN:2 * _RNN]
    g_g = gates[:, 2 * _RNN:3 * _RNN]
    o_g = gates[:, 3 * _RNN:4 * _RNN]
    c_new = jax.nn.sigmoid(f_g) * c_i + jax.nn.sigmoid(i_g) * jnp.tanh(g_g)
    h_new = jax.nn.sigmoid(o_g) * jnp.tanh(c_new)

    z = bdot(h_new, W_pi_ref[...]) + b_pi_ref[...]
    z = z - jnp.max(z, axis=1, keepdims=True)
    ez = jnp.exp(z)
    pi_ref[...] = ez / jnp.sum(ez, axis=1, keepdims=True)
    mu_ref[...] = bdot(h_new, W_mu_ref[...]) + b_mu_ref[...]
    sig_ref[...] = jnp.exp(bdot(h_new, W_sig_ref[...]) + b_sig_ref[...])
    h_out_ref[...] = h_new
    c_out_ref[...] = c_new


@jax.jit
def kernel(nodes, vis, h, c, W_in, b_in, W_rl, b_rl, W_rlv, b_rlv, W_score, b_score,
           W_mode, b_mode, W_ih, b_ih, W_hh, b_hh, W_pi, b_pi, W_mu, b_mu, W_sig, b_sig):
    n = nodes.shape[0]
    grid = (n // _BI,)
    kr = _K * _RNN

    full = lambda arr: pl.BlockSpec(arr.shape, lambda i: (0,) * arr.ndim)
    rows = lambda arr: pl.BlockSpec((_BI,) + arr.shape[1:], lambda i: (i,) + (0,) * (arr.ndim - 1))

    out_shapes = (
        jax.ShapeDtypeStruct((n, _G), _f32),
        jax.ShapeDtypeStruct((n, _G * _OUT), _f32),
        jax.ShapeDtypeStruct((n, _G * _OUT), _f32),
        jax.ShapeDtypeStruct((n, _RNN), _f32),
        jax.ShapeDtypeStruct((n, _RNN), _f32),
    )
    out_specs = tuple(rows(o) for o in out_shapes)

    args = (nodes, vis, h, c,
            W_in, b_in, W_rl, b_rl, W_rlv, b_rlv,
            W_score, W_mode, b_mode,
            W_ih, b_ih, W_hh, b_hh,
            W_pi, b_pi, W_mu, b_mu, W_sig, b_sig)
    in_specs = [full(nodes), rows(vis), rows(h), rows(c)] + [full(a) for a in args[4:]]

    pi, mu, sig, h_new, c_new = pl.pallas_call(
        _social_lstm_kernel,
        grid=grid,
        in_specs=in_specs,
        out_specs=out_specs,
        out_shape=out_shapes,
        scratch_shapes=[pltpu.VMEM((7, 2 * kr), _bf16)],
    )(*args)

    return pi, mu.reshape(n, _G, _OUT), sig.reshape(n, _G, _OUT), h_new, c_new
